# baseline (speedup 1.0000x reference)
import jax
import jax.numpy as jnp
from jax.experimental import pallas as pl
from jax.experimental.pallas import tpu as pltpu


def _make_body(P, H, C, chunks):
    half = H // 2
    quarter = half // 2

    def body(ch_ref, x_ref, o_ref, pos_ref):
        i = pl.program_id(0)
        j = jax.lax.broadcasted_iota(jnp.int32, (1, quarter), 1)
        omega = 1.0 / (10000.0 ** (j.astype(jnp.float32) / float(quarter)))

        sub = 1
        while sub * sub < P:
            sub *= 2

        @pl.when(i == 0)
        def _():
            if sub * sub == P:
                t = jax.lax.broadcasted_iota(jnp.int32, (sub, quarter), 0)
                t = t.astype(jnp.float32)
                ang_a = (t * float(sub)) * omega
                ang_b = t * omega
                sa_all, ca_all = jnp.sin(ang_a), jnp.cos(ang_a)
                sb, cb = jnp.sin(ang_b), jnp.cos(ang_b)
                for a in range(sub):
                    sa = sa_all[a:a + 1, :]
                    ca = ca_all[a:a + 1, :]
                    rows = pl.ds(a * sub, sub)
                    pos_ref[rows, :quarter] = sa * cb + ca * sb
                    pos_ref[rows, quarter:] = ca * cb - sa * sb
            else:
                p = jax.lax.broadcasted_iota(jnp.int32, (P, quarter), 0)
                ang = p.astype(jnp.float32) * omega
                pos_ref[:, :quarter] = jnp.sin(ang)
                pos_ref[:, quarter:] = jnp.cos(ang)

        for k in range(chunks):
            c = (i * chunks + k) % C
            ch = ch_ref[c].astype(jnp.float32)
            ang_c = ch * omega
            ch_row = jnp.concatenate([jnp.sin(ang_c), jnp.cos(ang_c)], axis=1)
            rows = pl.ds(k * P, P)
            xb = x_ref[0, rows, :]
            o_ref[0, rows, :half] = xb[:, :half] + ch_row
            o_ref[0, rows, half:] = xb[:, half:] + pos_ref[:, :]

    return body


def kernel(x, channels):
    B, CP, H = x.shape
    C = channels.shape[0]
    if not C:
        return x
    P = CP // C
    chunks = 5
    rows = chunks * P
    n = (B * CP) // rows
    xf = x.reshape(n, rows, H)
    out = pl.pallas_call(
        _make_body(P, H, C, chunks),
        grid=(n,),
        in_specs=[
            pl.BlockSpec(memory_space=pltpu.SMEM),
            pl.BlockSpec((1, rows, H), lambda i: (i, 0, 0)),
        ],
        out_specs=pl.BlockSpec((1, rows, H), lambda i: (i, 0, 0)),
        out_shape=jax.ShapeDtypeStruct(xf.shape, x.dtype),
        scratch_shapes=[pltpu.VMEM((P, H // 2), jnp.float32)],
    )(channels, xf)
    return out.reshape(x.shape)
